# baseline (speedup 1.0000x reference)
import jax
import jax.numpy as jnp
from jax.experimental import pallas as pl

N = 10000
D = 128
A = 4
L = 5


def _swish(v):
    return v * jax.nn.sigmoid(v)


def _final_mix_body(acc_ref, h_ref, na_ref, pp1W_ref, pp1V_ref, pp2W_ref, pp2V_ref, o_ref):
    h = acc_ref[...] + (0.7 ** 6) * h_ref[...]
    na = na_ref[...]
    g = h @ pp1W_ref[...] * (na @ pp1V_ref[...])
    g = _swish(g)
    o_ref[...] = (g @ pp2W_ref[...]) * (na @ pp2V_ref[...])


def kernel(x, pos, edge_index, additional_message_features, batch, epoch_step, embW, embV, msgW1, msgV1, msgW2, msgV2, updW1, updV1, updW2, updV2, pp1W, pp1V, pp2W, pp2V):
    amf = additional_message_features
    src, dst = edge_index[0], edge_index[1]
    rel = pos[src] - pos[dst]
    r = rel / (jnp.linalg.norm(rel, axis=-1, keepdims=True) + 1e-9)
    E = src.shape[0]
    edge_attr = jnp.concatenate([jnp.ones((E, 1), x.dtype), jnp.sqrt(3.0) * r], axis=-1)
    sums = jax.ops.segment_sum(edge_attr, dst, num_segments=N)
    cnts = jax.ops.segment_sum(jnp.ones((E, 1), x.dtype), dst, num_segments=N)
    node_attr = sums / jnp.maximum(cnts, 1.0)
    node_attr = node_attr.at[:, 0].set(1.0)

    alpha = 0.3
    coeffs = [alpha * (1 - alpha) ** i for i in range(6)] + [(1 - alpha) ** 6]
    h = (x @ embW) * (node_attr @ embV)
    acc = coeffs[0] * h
    for i in range(L):
        W1 = msgW1[i]
        Hd = h @ W1[:D]
        Hs = h @ W1[D:2 * D]
        pre = Hd[dst] + Hs[src] + amf * W1[2 * D][None, :]
        gate1 = edge_attr @ msgV1[i]
        m = _swish(pre * gate1)
        m = _swish((m @ msgW2[i]) * (edge_attr @ msgV2[i]))
        agg = jax.ops.segment_sum(m, dst, num_segments=N)
        U1 = updW1[i]
        t = h @ U1[:D] + agg @ U1[D:]
        u = _swish(t * (node_attr @ updV1[i]))
        u = (u @ updW2[i]) * (node_attr @ updV2[i])
        h = 0.7 * h + 0.3 * u
        acc = acc + coeffs[i + 1] * h

    out = pl.pallas_call(
        _final_mix_body,
        out_shape=jax.ShapeDtypeStruct((N, D), x.dtype),
    )(acc, h, node_attr, pp1W, pp1V, pp2W, pp2V)
    return out


# speedup vs baseline: 1.4913x; 1.4913x over previous
import functools

import jax
import jax.numpy as jnp
from jax import lax
from jax.experimental import pallas as pl
from jax.experimental.pallas import tpu as pltpu
from jax.experimental.pallas import tpu_sc as plsc

N = 10000
E = 160000
D = 128
A = 4
L = 5

NC = 2
NS = 16
NW = NC * NS
CH = 128
NCHUNK = 40
EP = NW * NCHUNK * CH
NP = 10240
ROWS_PER_SUB = NP // NS

_mesh = plsc.VectorSubcoreMesh(core_axis_name="c", subcore_axis_name="s")


def _gather_add_body(t1_hbm, t2_hbm, i1_hbm, i2_hbm, out_hbm,
                     i1_v, i2_v, rows_a, rows_b, sem_a, sem_b):
    wid = lax.axis_index("s") * NC + lax.axis_index("c")
    pltpu.sync_copy(i1_hbm.at[wid], i1_v)
    pltpu.sync_copy(i2_hbm.at[wid], i2_v)
    base = wid * (NCHUNK * CH)

    def chunk(j, buf):
        pltpu.async_copy(t1_hbm.at[buf_i1(j)], buf, sem_a).wait()
        pltpu.async_copy(t2_hbm.at[buf_i2(j)], buf, sem_b, add=True).wait()
        pltpu.sync_copy(buf, out_hbm.at[pl.ds(base + j * CH, CH)])

    def buf_i1(j):
        return i1_v.at[j]

    def buf_i2(j):
        return i2_v.at[j]

    def body(g, _):
        chunk(2 * g, rows_a)
        chunk(2 * g + 1, rows_b)
        return 0

    lax.fori_loop(0, NCHUNK // 2, body, 0)


def _sc_gather_add(t1, t2, i1, i2, dt):
    f = functools.partial(
        pl.kernel,
        out_type=jax.ShapeDtypeStruct((EP, dt), jnp.float32),
        mesh=_mesh,
        compiler_params=pltpu.CompilerParams(use_tc_tiling_on_sc=(dt == 128)),
        scratch_types=[
            pltpu.VMEM((NCHUNK, CH), jnp.int32),
            pltpu.VMEM((NCHUNK, CH), jnp.int32),
            pltpu.VMEM((CH, dt), jnp.float32),
            pltpu.VMEM((CH, dt), jnp.float32),
            pltpu.SemaphoreType.DMA,
            pltpu.SemaphoreType.DMA,
        ],
    )(_gather_add_body)
    return f(t1, t2, i1, i2)


def _scatter_body(dt, x_hbm, i_hbm, z_hbm, out_hbm, acc_sh, i_v, rows_v):
    c = lax.axis_index("c")
    s = lax.axis_index("s")
    wid = s * NC + c

    pltpu.sync_copy(z_hbm.at[pl.ds(s * ROWS_PER_SUB, ROWS_PER_SUB)],
                    acc_sh.at[pl.ds(s * ROWS_PER_SUB, ROWS_PER_SUB)])
    plsc.subcore_barrier()

    pltpu.sync_copy(i_hbm.at[wid], i_v)
    base = wid * (NCHUNK * CH)

    def body(j, _):
        pltpu.sync_copy(x_hbm.at[pl.ds(base + j * CH, CH)], rows_v)
        pltpu.sync_copy(rows_v, acc_sh.at[i_v.at[j]], add=True)
        return 0

    lax.fori_loop(0, NCHUNK, body, 0)
    plsc.subcore_barrier()
    pltpu.sync_copy(acc_sh.at[pl.ds(s * ROWS_PER_SUB, ROWS_PER_SUB)],
                    out_hbm.at[c, pl.ds(s * ROWS_PER_SUB, ROWS_PER_SUB)])


def _sc_segment_sum(x, idx3, dt):
    f = functools.partial(
        pl.kernel,
        out_type=jax.ShapeDtypeStruct((NC, NP, dt), jnp.float32),
        mesh=_mesh,
        compiler_params=pltpu.CompilerParams(use_tc_tiling_on_sc=(dt == 128)),
        scratch_types=[
            pltpu.VMEM_SHARED((NP, dt), jnp.float32),
            pltpu.VMEM((NCHUNK, CH), jnp.int32),
            pltpu.VMEM((CH, dt), jnp.float32),
        ],
    )(functools.partial(_scatter_body, dt))
    return f(x, idx3, jnp.zeros((NP, dt), jnp.float32))


def _swish(v):
    return v * jax.nn.sigmoid(v)


def kernel(x, pos, edge_index, additional_message_features, batch, epoch_step,
           embW, embV, msgW1, msgV1, msgW2, msgV2, updW1, updV1, updW2, updV2,
           pp1W, pp1V, pp2W, pp2V):
    amf = additional_message_features
    src, dst = edge_index[0], edge_index[1]
    pad = EP - E
    srcp = jnp.pad(src, (0, pad)).reshape(NW, NCHUNK, CH)
    dstp = jnp.pad(dst, (0, pad)).reshape(NW, NCHUNK, CH)

    pos16 = jnp.zeros((N, 16), jnp.float32).at[:, :3].set(pos)
    rel16 = _sc_gather_add(pos16, -pos16, srcp, dstp, 16)
    rel = rel16[:E, :3]
    r = rel / (jnp.linalg.norm(rel, axis=-1, keepdims=True) + 1e-9)
    edge_attr = jnp.concatenate([jnp.ones((E, 1), jnp.float32),
                                 jnp.sqrt(3.0) * r], axis=-1)
    ea16 = jnp.zeros((EP, 16), jnp.float32)
    ea16 = ea16.at[:E, :4].set(edge_attr).at[:E, 4].set(1.0)
    na_parts = _sc_segment_sum(ea16, dstp, 16)
    na_sum = na_parts[0, :N] + na_parts[1, :N]
    cnt = jnp.maximum(na_sum[:, 4:5], 1.0)
    node_attr = na_sum[:, :4] / cnt
    node_attr = node_attr.at[:, 0].set(1.0)

    alpha = 0.3
    coeffs = [alpha * (1 - alpha) ** i for i in range(6)] + [(1 - alpha) ** 6]
    h = (x @ embW) * (node_attr @ embV)
    acc = coeffs[0] * h
    gate_cache = edge_attr
    for i in range(L):
        W1 = msgW1[i]
        Hd = h @ W1[:D]
        Hs = h @ W1[D:2 * D]
        pre = _sc_gather_add(Hd, Hs, dstp, srcp, D)[:E]
        pre = pre + amf * W1[2 * D][None, :]
        m = _swish(pre * (gate_cache @ msgV1[i]))
        m = _swish((m @ msgW2[i]) * (gate_cache @ msgV2[i]))
        m_p = jnp.pad(m, ((0, pad), (0, 0)))
        agg_parts = _sc_segment_sum(m_p, dstp, D)
        agg = agg_parts[0, :N] + agg_parts[1, :N]
        U1 = updW1[i]
        t = h @ U1[:D] + agg @ U1[D:]
        u = _swish(t * (node_attr @ updV1[i]))
        u = (u @ updW2[i]) * (node_attr @ updV2[i])
        h = 0.7 * h + 0.3 * u
        acc = acc + coeffs[i + 1] * h

    h = acc + coeffs[6] * h
    g = _swish((h @ pp1W) * (node_attr @ pp1V))
    return (g @ pp2W) * (node_attr @ pp2V)


# speedup vs baseline: 2.0453x; 1.3714x over previous
import functools

import jax
import jax.numpy as jnp
from jax import lax
from jax.experimental import pallas as pl
from jax.experimental.pallas import tpu as pltpu
from jax.experimental.pallas import tpu_sc as plsc

N = 10000
E = 160000
D = 128
A = 4
L = 5

NC = 2
NS = 16
NW = NC * NS
CH = 128
NCHUNK = 40
EP = NW * NCHUNK * CH
NP = 10240
ROWS_PER_SUB = NP // NS

_mesh = plsc.VectorSubcoreMesh(core_axis_name="c", subcore_axis_name="s")


_RING = 4


def _gather_add_body(t1_hbm, t2_hbm, i1_hbm, i2_hbm, out_hbm,
                     i1_v, i2_v, *rest):
    bufs = rest[:_RING]
    sems_a = rest[_RING:2 * _RING]
    sems_b = rest[2 * _RING:3 * _RING]
    sems_c = rest[3 * _RING:4 * _RING]
    wid = lax.axis_index("s") * NC + lax.axis_index("c")
    pltpu.sync_copy(i1_hbm.at[wid], i1_v)
    pltpu.sync_copy(i2_hbm.at[wid], i2_v)
    base = wid * (NCHUNK * CH)

    g1 = [None] * _RING
    g2 = [None] * _RING
    st = [None] * _RING
    for j in range(NCHUNK + 2):
        if j < NCHUNK:
            r = j % _RING
            if st[r] is not None:
                st[r].wait()
            g1[r] = pltpu.async_copy(t1_hbm.at[i1_v.at[j]], bufs[r], sems_a[r])
        if 1 <= j < NCHUNK + 1:
            jj = j - 1
            r = jj % _RING
            g1[r].wait()
            g2[r] = pltpu.async_copy(t2_hbm.at[i2_v.at[jj]], bufs[r],
                                     sems_b[r], add=True)
        if j >= 2:
            jj = j - 2
            r = jj % _RING
            g2[r].wait()
            st[r] = pltpu.async_copy(
                bufs[r], out_hbm.at[pl.ds(base + jj * CH, CH)], sems_c[r])
    for j in range(NCHUNK - _RING, NCHUNK):
        st[j % _RING].wait()


def _sc_gather_add(t1, t2, i1, i2, dt):
    f = functools.partial(
        pl.kernel,
        out_type=jax.ShapeDtypeStruct((EP, dt), jnp.float32),
        mesh=_mesh,
        compiler_params=pltpu.CompilerParams(use_tc_tiling_on_sc=(dt == 128)),
        scratch_types=[
            pltpu.VMEM((NCHUNK, CH), jnp.int32),
            pltpu.VMEM((NCHUNK, CH), jnp.int32),
        ] + [pltpu.VMEM((CH, dt), jnp.float32)] * _RING
          + [pltpu.SemaphoreType.DMA] * (3 * _RING),
    )(_gather_add_body)
    return f(t1, t2, i1, i2)


_SRING = 2


def _scatter_body(dt, x_hbm, i_hbm, z_hbm, out_hbm, acc_sh, i_v, *rest):
    bufs = rest[:_SRING]
    sems = rest[_SRING:2 * _SRING]
    sems_w = rest[2 * _SRING:3 * _SRING]
    c = lax.axis_index("c")
    s = lax.axis_index("s")
    wid = s * NC + c

    pltpu.sync_copy(z_hbm.at[pl.ds(s * ROWS_PER_SUB, ROWS_PER_SUB)],
                    acc_sh.at[pl.ds(s * ROWS_PER_SUB, ROWS_PER_SUB)])
    pltpu.sync_copy(i_hbm.at[wid], i_v)
    plsc.subcore_barrier()
    base = wid * (NCHUNK * CH)

    rd = [None] * _SRING
    wr = [None] * _SRING
    for j in range(NCHUNK + 1):
        if j < NCHUNK:
            r = j % _SRING
            if wr[r] is not None:
                wr[r].wait()
            rd[r] = pltpu.async_copy(
                x_hbm.at[pl.ds(base + j * CH, CH)], bufs[r], sems[r])
        if j >= 1:
            jj = j - 1
            r = jj % _SRING
            rd[r].wait()
            wr[r] = pltpu.async_copy(bufs[r], acc_sh.at[i_v.at[jj]],
                                     sems_w[r], add=True)
    for j in range(NCHUNK - _SRING, NCHUNK):
        if wr[j % _SRING] is not None:
            wr[j % _SRING].wait()
    plsc.subcore_barrier()
    pltpu.sync_copy(acc_sh.at[pl.ds(s * ROWS_PER_SUB, ROWS_PER_SUB)],
                    out_hbm.at[c, pl.ds(s * ROWS_PER_SUB, ROWS_PER_SUB)])


def _sc_segment_sum(x, idx3, dt):
    f = functools.partial(
        pl.kernel,
        out_type=jax.ShapeDtypeStruct((NC, NP, dt), jnp.float32),
        mesh=_mesh,
        compiler_params=pltpu.CompilerParams(use_tc_tiling_on_sc=(dt == 128)),
        scratch_types=[
            pltpu.VMEM_SHARED((NP, dt), jnp.float32),
            pltpu.VMEM((NCHUNK, CH), jnp.int32),
        ] + [pltpu.VMEM((CH, dt), jnp.float32)] * _SRING
          + [pltpu.SemaphoreType.DMA] * (2 * _SRING),
    )(functools.partial(_scatter_body, dt))
    return f(x, idx3, jnp.zeros((NP, dt), jnp.float32))


def _swish(v):
    return v * jax.nn.sigmoid(v)


def kernel(x, pos, edge_index, additional_message_features, batch, epoch_step,
           embW, embV, msgW1, msgV1, msgW2, msgV2, updW1, updV1, updW2, updV2,
           pp1W, pp1V, pp2W, pp2V):
    amf = additional_message_features
    src, dst = edge_index[0], edge_index[1]
    pad = EP - E
    srcp = jnp.pad(src, (0, pad)).reshape(NW, NCHUNK, CH)
    dstp = jnp.pad(dst, (0, pad)).reshape(NW, NCHUNK, CH)

    pos16 = jnp.zeros((N, 16), jnp.float32).at[:, :3].set(pos)
    rel16 = _sc_gather_add(pos16, -pos16, srcp, dstp, 16)
    rel = rel16[:E, :3]
    r = rel / (jnp.linalg.norm(rel, axis=-1, keepdims=True) + 1e-9)
    edge_attr = jnp.concatenate([jnp.ones((E, 1), jnp.float32),
                                 jnp.sqrt(3.0) * r], axis=-1)
    ea16 = jnp.zeros((EP, 16), jnp.float32)
    ea16 = ea16.at[:E, :4].set(edge_attr).at[:E, 4].set(1.0)
    na_parts = _sc_segment_sum(ea16, dstp, 16)
    na_sum = na_parts[0, :N] + na_parts[1, :N]
    cnt = jnp.maximum(na_sum[:, 4:5], 1.0)
    node_attr = na_sum[:, :4] / cnt
    node_attr = node_attr.at[:, 0].set(1.0)

    alpha = 0.3
    coeffs = [alpha * (1 - alpha) ** i for i in range(6)] + [(1 - alpha) ** 6]
    h = (x @ embW) * (node_attr @ embV)
    acc = coeffs[0] * h
    gate_cache = edge_attr
    for i in range(L):
        W1 = msgW1[i]
        Hd = h @ W1[:D]
        Hs = h @ W1[D:2 * D]
        pre = _sc_gather_add(Hd, Hs, dstp, srcp, D)[:E]
        pre = pre + amf * W1[2 * D][None, :]
        m = _swish(pre * (gate_cache @ msgV1[i]))
        m = _swish((m @ msgW2[i]) * (gate_cache @ msgV2[i]))
        m_p = jnp.pad(m, ((0, pad), (0, 0)))
        agg_parts = _sc_segment_sum(m_p, dstp, D)
        agg = agg_parts[0, :N] + agg_parts[1, :N]
        U1 = updW1[i]
        t = h @ U1[:D] + agg @ U1[D:]
        u = _swish(t * (node_attr @ updV1[i]))
        u = (u @ updW2[i]) * (node_attr @ updV2[i])
        h = 0.7 * h + 0.3 * u
        acc = acc + coeffs[i + 1] * h

    h = acc + coeffs[6] * h
    g = _swish((h @ pp1W) * (node_attr @ pp1V))
    return (g @ pp2W) * (node_attr @ pp2V)


# speedup vs baseline: 2.0457x; 1.0002x over previous
import functools

import jax
import jax.numpy as jnp
from jax import lax
from jax.experimental import pallas as pl
from jax.experimental.pallas import tpu as pltpu
from jax.experimental.pallas import tpu_sc as plsc

N = 10000
E = 160000
D = 128
A = 4
L = 5

NC = 2
NS = 16
NW = NC * NS
CH = 128
NCHUNK = 40
EP = NW * NCHUNK * CH
NP = 10240
ROWS_PER_SUB = NP // NS

_mesh = plsc.VectorSubcoreMesh(core_axis_name="c", subcore_axis_name="s")


_RING = 6
_LAG = 2


def _gather_add_body(t1_hbm, t2_hbm, i1_hbm, i2_hbm, out_hbm,
                     i1_v, i2_v, *rest):
    bufs = rest[:_RING]
    sems_a = rest[_RING:2 * _RING]
    sems_b = rest[2 * _RING:3 * _RING]
    sems_c = rest[3 * _RING:4 * _RING]
    wid = lax.axis_index("s") * NC + lax.axis_index("c")
    pltpu.sync_copy(i1_hbm.at[wid], i1_v)
    pltpu.sync_copy(i2_hbm.at[wid], i2_v)
    base = wid * (NCHUNK * CH)

    g1 = [None] * _RING
    g2 = [None] * _RING
    st = [None] * _RING
    for j in range(NCHUNK + 2 * _LAG):
        if j < NCHUNK:
            r = j % _RING
            if st[r] is not None:
                st[r].wait()
            g1[r] = pltpu.async_copy(t1_hbm.at[i1_v.at[j]], bufs[r], sems_a[r])
        if _LAG <= j < NCHUNK + _LAG:
            jj = j - _LAG
            r = jj % _RING
            g1[r].wait()
            g2[r] = pltpu.async_copy(t2_hbm.at[i2_v.at[jj]], bufs[r],
                                     sems_b[r], add=True)
        if j >= 2 * _LAG:
            jj = j - 2 * _LAG
            r = jj % _RING
            g2[r].wait()
            st[r] = pltpu.async_copy(
                bufs[r], out_hbm.at[pl.ds(base + jj * CH, CH)], sems_c[r])
    for j in range(NCHUNK - _RING, NCHUNK):
        st[j % _RING].wait()


def _sc_gather_add(t1, t2, i1, i2, dt):
    f = functools.partial(
        pl.kernel,
        out_type=jax.ShapeDtypeStruct((EP, dt), jnp.float32),
        mesh=_mesh,
        compiler_params=pltpu.CompilerParams(use_tc_tiling_on_sc=(dt == 128)),
        scratch_types=[
            pltpu.VMEM((NCHUNK, CH), jnp.int32),
            pltpu.VMEM((NCHUNK, CH), jnp.int32),
        ] + [pltpu.VMEM((CH, dt), jnp.float32)] * _RING
          + [pltpu.SemaphoreType.DMA] * (3 * _RING),
    )(_gather_add_body)
    return f(t1, t2, i1, i2)


_SRING = 2


def _scatter_body(dt, x_hbm, i_hbm, z_hbm, out_hbm, acc_sh, i_v, *rest):
    bufs = rest[:_SRING]
    sems = rest[_SRING:2 * _SRING]
    sems_w = rest[2 * _SRING:3 * _SRING]
    c = lax.axis_index("c")
    s = lax.axis_index("s")
    wid = s * NC + c

    pltpu.sync_copy(z_hbm.at[pl.ds(s * ROWS_PER_SUB, ROWS_PER_SUB)],
                    acc_sh.at[pl.ds(s * ROWS_PER_SUB, ROWS_PER_SUB)])
    pltpu.sync_copy(i_hbm.at[wid], i_v)
    plsc.subcore_barrier()
    base = wid * (NCHUNK * CH)

    rd = [None] * _SRING
    wr = [None] * _SRING
    for j in range(NCHUNK + 1):
        if j < NCHUNK:
            r = j % _SRING
            if wr[r] is not None:
                wr[r].wait()
            rd[r] = pltpu.async_copy(
                x_hbm.at[pl.ds(base + j * CH, CH)], bufs[r], sems[r])
        if j >= 1:
            jj = j - 1
            r = jj % _SRING
            rd[r].wait()
            wr[r] = pltpu.async_copy(bufs[r], acc_sh.at[i_v.at[jj]],
                                     sems_w[r], add=True)
    for j in range(NCHUNK - _SRING, NCHUNK):
        if wr[j % _SRING] is not None:
            wr[j % _SRING].wait()
    plsc.subcore_barrier()
    pltpu.sync_copy(acc_sh.at[pl.ds(s * ROWS_PER_SUB, ROWS_PER_SUB)],
                    out_hbm.at[c, pl.ds(s * ROWS_PER_SUB, ROWS_PER_SUB)])


def _sc_segment_sum(x, idx3, dt):
    f = functools.partial(
        pl.kernel,
        out_type=jax.ShapeDtypeStruct((NC, NP, dt), jnp.float32),
        mesh=_mesh,
        compiler_params=pltpu.CompilerParams(use_tc_tiling_on_sc=(dt == 128)),
        scratch_types=[
            pltpu.VMEM_SHARED((NP, dt), jnp.float32),
            pltpu.VMEM((NCHUNK, CH), jnp.int32),
        ] + [pltpu.VMEM((CH, dt), jnp.float32)] * _SRING
          + [pltpu.SemaphoreType.DMA] * (2 * _SRING),
    )(functools.partial(_scatter_body, dt))
    return f(x, idx3, jnp.zeros((NP, dt), jnp.float32))


def _swish(v):
    return v * jax.nn.sigmoid(v)


def kernel(x, pos, edge_index, additional_message_features, batch, epoch_step,
           embW, embV, msgW1, msgV1, msgW2, msgV2, updW1, updV1, updW2, updV2,
           pp1W, pp1V, pp2W, pp2V):
    amf = additional_message_features
    src, dst = edge_index[0], edge_index[1]
    pad = EP - E
    srcp = jnp.pad(src, (0, pad)).reshape(NW, NCHUNK, CH)
    dstp = jnp.pad(dst, (0, pad)).reshape(NW, NCHUNK, CH)

    pos16 = jnp.zeros((N, 16), jnp.float32).at[:, :3].set(pos)
    rel16 = _sc_gather_add(pos16, -pos16, srcp, dstp, 16)
    rel = rel16[:E, :3]
    r = rel / (jnp.linalg.norm(rel, axis=-1, keepdims=True) + 1e-9)
    edge_attr = jnp.concatenate([jnp.ones((E, 1), jnp.float32),
                                 jnp.sqrt(3.0) * r], axis=-1)
    ea16 = jnp.zeros((EP, 16), jnp.float32)
    ea16 = ea16.at[:E, :4].set(edge_attr).at[:E, 4].set(1.0)
    na_parts = _sc_segment_sum(ea16, dstp, 16)
    na_sum = na_parts[0, :N] + na_parts[1, :N]
    cnt = jnp.maximum(na_sum[:, 4:5], 1.0)
    node_attr = na_sum[:, :4] / cnt
    node_attr = node_attr.at[:, 0].set(1.0)

    alpha = 0.3
    coeffs = [alpha * (1 - alpha) ** i for i in range(6)] + [(1 - alpha) ** 6]
    h = (x @ embW) * (node_attr @ embV)
    acc = coeffs[0] * h
    gate_cache = edge_attr
    for i in range(L):
        W1 = msgW1[i]
        Hd = h @ W1[:D]
        Hs = h @ W1[D:2 * D]
        pre = _sc_gather_add(Hd, Hs, dstp, srcp, D)[:E]
        pre = pre + amf * W1[2 * D][None, :]
        m = _swish(pre * (gate_cache @ msgV1[i]))
        m = _swish((m @ msgW2[i]) * (gate_cache @ msgV2[i]))
        m_p = jnp.pad(m, ((0, pad), (0, 0)))
        agg_parts = _sc_segment_sum(m_p, dstp, D)
        agg = agg_parts[0, :N] + agg_parts[1, :N]
        U1 = updW1[i]
        t = h @ U1[:D] + agg @ U1[D:]
        u = _swish(t * (node_attr @ updV1[i]))
        u = (u @ updW2[i]) * (node_attr @ updV2[i])
        h = 0.7 * h + 0.3 * u
        acc = acc + coeffs[i + 1] * h

    h = acc + coeffs[6] * h
    g = _swish((h @ pp1W) * (node_attr @ pp1V))
    return (g @ pp2W) * (node_attr @ pp2V)
